# Initial kernel scaffold; baseline (speedup 1.0000x reference)
import jax
import jax.numpy as jnp
from jax.experimental import pallas as pl


def kernel(features_1, edge_index_1, batch_1, features_2, edge_index_2, batch_2, params):
    raise NotImplementedError("write your pallas kernel here")


# speedup vs baseline: 2.3799x; 2.3799x over previous
import functools

import jax
import jax.numpy as jnp
from jax import lax
from jax.experimental import pallas as pl
from jax.experimental.pallas import tpu as pltpu
from jax.experimental.pallas import tpu_sc as plsc

N = 10000
E = 160000
B = 128
RB = 512
NBLK = 20
NPAD = NBLK * RB

SC_K = 128
E_CHUNKS = 1280
E_PAD = E_CHUNKS * SC_K
ACC_ROWS = 10112
DUMP_ROW = N
ZROWS = 632
CPROWS = 624

@functools.cache
def _make_segsum(split_edges: bool):
    rows_per_tile = 40 if split_edges else 80

    @functools.partial(
        pl.kernel,
        out_type=[jax.ShapeDtypeStruct((N, 128), jnp.float32)] * 2,
        mesh=plsc.VectorSubcoreMesh(core_axis_name="c", subcore_axis_name="s"),
        scratch_types=[
            pltpu.VMEM((rows_per_tile, SC_K), jnp.int32),
            pltpu.VMEM((rows_per_tile, SC_K), jnp.int32),
            pltpu.VMEM((SC_K, 128), jnp.float32),
            pltpu.VMEM_SHARED((ACC_ROWS, 128), jnp.float32),
            pltpu.SemaphoreType.DMA,
        ],
    )
    def segsum(t0, t1, src2, dst2, zer, out0, out1, srcv, dstv, rowsv, acc, sem):
        c = lax.axis_index("c")
        s = lax.axis_index("s")
        pltpu.sync_copy(zer, acc.at[pl.ds(s * ZROWS, ZROWS)])
        plsc.subcore_barrier()

        if split_edges:
            row_base = (c * 16 + s) * rows_per_tile
        else:
            row_base = s * rows_per_tile

        def run(table, out):
            pltpu.sync_copy(src2.at[pl.ds(row_base, rows_per_tile)], srcv)
            pltpu.sync_copy(dst2.at[pl.ds(row_base, rows_per_tile)], dstv)

            def body(j, carry):
                pltpu.async_copy(table.at[srcv.at[j]], rowsv, sem).wait()
                pltpu.sync_copy(rowsv, acc.at[dstv.at[j]], add=True)
                return carry

            lax.fori_loop(0, rows_per_tile, body, 0, unroll=False)
            plsc.subcore_barrier()
            pltpu.sync_copy(acc.at[pl.ds(s * CPROWS, CPROWS)],
                            out.at[pl.ds(s * CPROWS, CPROWS)])

            @pl.when(s == 0)
            def _():
                pltpu.sync_copy(acc.at[pl.ds(16 * CPROWS, N - 16 * CPROWS)],
                                out.at[pl.ds(16 * CPROWS, N - 16 * CPROWS)])

        @pl.when(c == 0)
        def _():
            run(t0, out0)

        @pl.when(c == 1)
        def _():
            run(t1, out1)

    return segsum


def _segsum_sc(t0, t1, src2, dst2, zer, split_edges):
    return _make_segsum(bool(split_edges))(t0, t1, src2, dst2, zer)


_segsum_dispatch = _segsum_sc


def _full(shape):
    nd = len(shape)
    return pl.BlockSpec(shape, lambda i, _nd=nd: (0,) * _nd)


def _full0(shape):
    nd = len(shape)
    return pl.BlockSpec(shape, lambda _nd=nd: (0,) * _nd)


def _rowmask(i):
    rows = i * RB + lax.broadcasted_iota(jnp.int32, (RB, 1), 0)
    return rows < N


def _sigmoid(x):
    return 1.0 / (1.0 + jnp.exp(-x))


def _gin_body_l1(x_ref, a0_ref, a1_ref, w1_ref, b1_ref, w2_ref, b2_ref,
                 eps_ref, y_ref, st_ref):
    i = pl.program_id(0)
    h = (1.0 + eps_ref[0, 0]) * x_ref[...] + a0_ref[...] + a1_ref[...]
    z = jnp.maximum(jnp.dot(h, w1_ref[...], preferred_element_type=jnp.float32)
                    + b1_ref[...], 0.0)
    y = jnp.dot(z, w2_ref[...], preferred_element_type=jnp.float32) + b2_ref[...]
    ym = jnp.where(_rowmask(i), y, 0.0)
    y_ref[...] = ym
    st = jnp.concatenate([jnp.sum(ym, axis=0, keepdims=True),
                          jnp.sum(ym * ym, axis=0, keepdims=True)], axis=0)

    @pl.when(i == 0)
    def _():
        st_ref[...] = st

    @pl.when(i > 0)
    def _():
        st_ref[...] += st


def _gin_body_l23(xa_ref, xb_ref, aa_ref, ab_ref, w1_ref, b1_ref, w2_ref,
                  b2_ref, eps_ref, y_ref, st_ref):
    i = pl.program_id(0)
    x = jnp.concatenate([xa_ref[...], xb_ref[...]], axis=1)
    agg = jnp.concatenate([aa_ref[...], ab_ref[...]], axis=1)
    h = (1.0 + eps_ref[0, 0]) * x + agg
    z = jnp.maximum(jnp.dot(h, w1_ref[...], preferred_element_type=jnp.float32)
                    + b1_ref[...], 0.0)
    y = jnp.dot(z, w2_ref[...], preferred_element_type=jnp.float32) + b2_ref[...]
    ym = jnp.where(_rowmask(i), y, 0.0)
    y_ref[...] = ym
    st = jnp.concatenate([jnp.sum(ym, axis=0, keepdims=True),
                          jnp.sum(ym * ym, axis=0, keepdims=True)], axis=0)

    @pl.when(i == 0)
    def _():
        st_ref[...] = st

    @pl.when(i > 0)
    def _():
        st_ref[...] += st


def _gin_matmul(x_or_halves, agg_pair, p, din):
    b1 = p["b1"].reshape(1, -1)
    b2 = p["b2"].reshape(1, -1)
    eps = p["eps"].reshape(1, 1)
    dout = p["W1"].shape[1]
    if din == 128:
        body = _gin_body_l1
        ins = [x_or_halves, agg_pair[0], agg_pair[1]]
        in_specs = [pl.BlockSpec((RB, 128), lambda i: (i, 0))] * 3
    else:
        body = _gin_body_l23
        ins = [x_or_halves[0], x_or_halves[1], agg_pair[0], agg_pair[1]]
        in_specs = [pl.BlockSpec((RB, 128), lambda i: (i, 0))] * 4
    in_specs += [_full(p["W1"].shape), _full((1, dout)),
                 _full(p["W2"].shape), _full((1, dout)), _full((1, 1))]
    y, st = pl.pallas_call(
        body,
        grid=(NBLK,),
        in_specs=in_specs,
        out_specs=[pl.BlockSpec((RB, dout), lambda i: (i, 0)),
                   pl.BlockSpec((2, dout), lambda i: (0, 0))],
        out_shape=[jax.ShapeDtypeStruct((N, dout), jnp.float32),
                   jax.ShapeDtypeStruct((2, dout), jnp.float32)],
    )(*ins, p["W1"], b1, p["W2"], b2, eps)
    return y, st


def _norm_body(y_ref, st_ref, g_ref, beta_ref, ha_ref, hb_ref):
    m = st_ref[0:1, :] / float(N)
    v = st_ref[1:2, :] / float(N) - m * m
    inv = lax.rsqrt(v + 1e-5)
    h = jnp.maximum((y_ref[...] - m) * inv * g_ref[...] + beta_ref[...], 0.0)
    ha_ref[...] = h[:, :128]
    hb_ref[...] = h[:, 128:]


def _gin_norm(y, st, p):
    dout = y.shape[1]
    return pl.pallas_call(
        _norm_body,
        grid=(NBLK,),
        in_specs=[pl.BlockSpec((RB, dout), lambda i: (i, 0)),
                  _full((2, dout)), _full((1, dout)), _full((1, dout))],
        out_specs=[pl.BlockSpec((RB, 128), lambda i: (i, 0))] * 2,
        out_shape=[jax.ShapeDtypeStruct((N, 128), jnp.float32)] * 2,
    )(y, st, p["g"].reshape(1, -1), p["beta"].reshape(1, -1))


def _att1_body(ha_ref, hb_ref, bt_ref, a1_ref, b1_ref, a2_ref, b2_ref,
               xg_ref, s_ref, cnt_ref):
    i = pl.program_id(0)
    h = jnp.concatenate([ha_ref[...], hb_ref[...]], axis=1)
    z = jnp.maximum(jnp.dot(h, a1_ref[...], preferred_element_type=jnp.float32)
                    + b1_ref[...], 0.0)
    a = jnp.tanh(jnp.dot(z, a2_ref[...], preferred_element_type=jnp.float32)
                 + b2_ref[...])
    xg = h * (1.0 + a)
    xgm = jnp.where(_rowmask(i), xg, 0.0)
    xg_ref[...] = xgm
    bt = bt_ref[...].reshape(1, RB)
    onehot_t = jnp.where(
        lax.broadcasted_iota(jnp.int32, (B, RB), 0) == bt, 1.0, 0.0)
    s_part = jnp.dot(onehot_t, xgm, preferred_element_type=jnp.float32)
    cnt_part = jnp.sum(onehot_t, axis=1, keepdims=True)
    cnt_part = jnp.broadcast_to(cnt_part, (B, 128))

    @pl.when(i == 0)
    def _():
        s_ref[...] = s_part
        cnt_ref[...] = cnt_part

    @pl.when(i > 0)
    def _():
        s_ref[...] += s_part
        cnt_ref[...] += cnt_part


def _att2_body(s_ref, cnt_ref, w_ref, tg_ref):
    cnt = cnt_ref[:, 0:1]
    mean = jnp.where(cnt > 0, s_ref[...] / jnp.maximum(cnt, 1.0), 0.0)
    tg_ref[...] = jnp.tanh(jnp.dot(mean, w_ref[...],
                                   preferred_element_type=jnp.float32))


def _att3_body(xg_ref, bt_ref, tg_ref, out_ref):
    i = pl.program_id(0)
    xgm = jnp.where(_rowmask(i), xg_ref[...], 0.0)
    bt = bt_ref[...].reshape(1, RB)
    onehot_t = jnp.where(
        lax.broadcasted_iota(jnp.int32, (B, RB), 0) == bt, 1.0, 0.0)
    tgrow = lax.dot_general(onehot_t, tg_ref[...], (((0,), (0,)), ((), ())),
                            preferred_element_type=jnp.float32)
    coefs = _sigmoid(jnp.sum(xgm * tgrow, axis=1, keepdims=True))
    px = coefs * xgm
    p_part = jnp.dot(onehot_t, px, preferred_element_type=jnp.float32)

    @pl.when(i == 0)
    def _():
        out_ref[...] = p_part

    @pl.when(i > 0)
    def _():
        out_ref[...] += p_part


def _att_pool(ha, hb, bt3, p):
    d = p["W"].shape[0]
    r = d // 4
    xg, s, cnt = pl.pallas_call(
        _att1_body,
        grid=(NBLK,),
        in_specs=[pl.BlockSpec((RB, 128), lambda i: (i, 0)),
                  pl.BlockSpec((RB, 128), lambda i: (i, 0)),
                  pl.BlockSpec((1, 1, RB), lambda i: (i, 0, 0)),
                  _full((d, r)), _full((1, r)), _full((r, d)), _full((1, d))],
        out_specs=[pl.BlockSpec((RB, d), lambda i: (i, 0)),
                   pl.BlockSpec((B, d), lambda i: (0, 0)),
                   pl.BlockSpec((B, 128), lambda i: (0, 0))],
        out_shape=[jax.ShapeDtypeStruct((N, d), jnp.float32),
                   jax.ShapeDtypeStruct((B, d), jnp.float32),
                   jax.ShapeDtypeStruct((B, 128), jnp.float32)],
    )(ha, hb, bt3, p["A1"], p["b1"].reshape(1, -1), p["A2"],
      p["b2"].reshape(1, -1))

    tg = pl.pallas_call(
        _att2_body,
        in_specs=[_full0((B, d)), _full0((B, 128)), _full0((d, d))],
        out_specs=_full0((B, d)),
        out_shape=jax.ShapeDtypeStruct((B, d), jnp.float32),
    )(s, cnt, p["W"])

    pooled = pl.pallas_call(
        _att3_body,
        grid=(NBLK,),
        in_specs=[pl.BlockSpec((RB, d), lambda i: (i, 0)),
                  pl.BlockSpec((1, 1, RB), lambda i: (i, 0, 0)),
                  _full((B, d))],
        out_specs=pl.BlockSpec((B, d), lambda i: (0, 0)),
        out_shape=jax.ShapeDtypeStruct((B, d), jnp.float32),
    )(xg, bt3, tg)
    return pooled


def _head_body(p1a_ref, p1b_ref, p2a_ref, p2b_ref, p3a_ref, p3b_ref,
               t1s1, t1sb1, t1s2, t1sb2, t1t1, t1tb1, t1t2, t1tb2,
               t2s1, t2sb1, t2s2, t2sb2, t2t1, t2tb1, t2t2, t2tb2,
               t3s1, t3sb1, t3s2, t3sb2, t3t1, t3tb1, t3t2, t3tb2,
               saa1, sab1, saa2, sab2, fcw, fcb, out_ref):
    def dot(a, b):
        return jnp.dot(a, b, preferred_element_type=jnp.float32)

    def tn(ea, eb, s1, sb1, s2, sb2, t1, tb1, t2, tb2):
        c = jnp.concatenate([ea, eb], axis=1)
        se = _sigmoid(dot(jnp.maximum(dot(c, s1[...]) + sb1[...], 0.0),
                          s2[...]) + sb2[...])
        sf = se * c + c
        h = jnp.maximum(dot(sf, t1[...]) + tb1[...], 0.0)
        return jnp.maximum(dot(h, t2[...]) + tb2[...], 0.0)

    s1 = tn(p1a_ref[...], p1b_ref[...], t1s1, t1sb1, t1s2, t1sb2,
            t1t1, t1tb1, t1t2, t1tb2)
    s2 = tn(p2a_ref[...], p2b_ref[...], t2s1, t2sb1, t2s2, t2sb2,
            t2t1, t2tb1, t2t2, t2tb2)
    s3 = tn(p3a_ref[...], p3b_ref[...], t3s1, t3sb1, t3s2, t3sb2,
            t3t1, t3tb1, t3t2, t3tb2)
    scores = jnp.concatenate([s3, s2, s1], axis=1)
    att = _sigmoid(dot(jnp.maximum(dot(scores, saa1[...]) + sab1[...], 0.0),
                       saa2[...]) + sab2[...])
    out_ref[...] = jnp.maximum(dot(att * scores + scores, fcw[...])
                               + fcb[...], 0.0)


def _head(p1, p2, p3, params):
    ins = list(p1) + list(p2) + list(p3)
    for key in ("tn1", "tn2", "tn3"):
        t = params[key]
        ins += [t["S1"], t["sb1"].reshape(1, -1), t["S2"], t["sb2"].reshape(1, -1),
                t["T1"], t["tb1"].reshape(1, -1), t["T2"], t["tb2"].reshape(1, -1)]
    ins += [params["saA1"], params["sab1"].reshape(1, -1),
            params["saA2"], params["sab2"].reshape(1, -1),
            params["fcW"], params["fcb"].reshape(1, -1)]
    return pl.pallas_call(
        _head_body,
        in_specs=[_full0(x.shape) for x in ins],
        out_specs=_full0((B, 128)),
        out_shape=jax.ShapeDtypeStruct((B, 128), jnp.float32),
    )(*ins)


def _edge_arrays(ei):
    src = jnp.concatenate([ei[0], jnp.zeros((E_PAD - E,), jnp.int32)])
    dst = jnp.concatenate([ei[1], jnp.full((E_PAD - E,), DUMP_ROW, jnp.int32)])
    return src.reshape(E_CHUNKS, SC_K), dst.reshape(E_CHUNKS, SC_K)


def _graph_levels(params, x, src2, dst2, zer, bt3):
    a0, a1 = _segsum_dispatch(x, x, src2, dst2, zer, True)
    y, st = _gin_matmul(x, (a0, a1), params["gin1"], 128)
    h1a, h1b = _gin_norm(y, st, params["gin1"])
    aa, ab = _segsum_dispatch(h1a, h1b, src2, dst2, zer, False)
    y, st = _gin_matmul((h1a, h1b), (aa, ab), params["gin2"], 256)
    h2a, h2b = _gin_norm(y, st, params["gin2"])
    aa, ab = _segsum_dispatch(h2a, h2b, src2, dst2, zer, False)
    y, st = _gin_matmul((h2a, h2b), (aa, ab), params["gin3"], 256)
    h3a, h3b = _gin_norm(y, st, params["gin3"])
    p1 = _att_pool(h1a, h1b, bt3, params["att1"])
    p2 = _att_pool(h2a, h2b, bt3, params["att2"])
    p3 = _att_pool(h3a, h3b, bt3, params["att3"])
    return p1, p2, p3


def kernel(features_1, edge_index_1, batch_1, features_2, edge_index_2,
           batch_2, params):
    zer = jnp.zeros((ZROWS, 128), jnp.float32)
    src2_1, dst2_1 = _edge_arrays(edge_index_1)
    src2_2, dst2_2 = _edge_arrays(edge_index_2)
    pad = jnp.full((NPAD - N,), 255, jnp.int32)
    bt3_1 = jnp.concatenate([batch_1, pad]).reshape(NBLK, 1, RB)
    bt3_2 = jnp.concatenate([batch_2, pad]).reshape(NBLK, 1, RB)

    p1a, p2a, p3a = _graph_levels(params, features_1, src2_1, dst2_1, zer, bt3_1)
    p1b, p2b, p3b = _graph_levels(params, features_2, src2_2, dst2_2, zer, bt3_2)

    return _head((p1a, p1b), (p2a, p2b), (p3a, p3b), params)
